# Initial kernel scaffold; baseline (speedup 1.0000x reference)
import jax
import jax.numpy as jnp
from jax.experimental import pallas as pl


def kernel(x_drug, x_protein, edge_index_dd, edge_index_dp, W_dd, W_dp, h_bias):
    raise NotImplementedError("write your pallas kernel here")


# speedup vs baseline: 2.3231x; 2.3231x over previous
import functools

import jax
import jax.numpy as jnp
from jax import lax
from jax.experimental import pallas as pl
from jax.experimental.pallas import tpu as pltpu
from jax.experimental.pallas import tpu_sc as plsc

N = 50000
D = 128
E = 300000
NROWS = 50176
DUMMY = 50000
CW = 16
NCH = D // CW
NS = 16
KJ = 147
EPAD = NS * KJ * 128
ZROWS = 392
TROWS = NROWS // NS
BR = 512
GRID = NROWS // BR

_mesh = plsc.VectorSubcoreMesh(core_axis_name="c", subcore_axis_name="s")
_sc_params = pltpu.CompilerParams(use_tc_tiling_on_sc=False)
f32 = jnp.float32


@functools.partial(
    pl.kernel,
    out_type=[jax.ShapeDtypeStruct((NROWS,), f32)] * 4,
    mesh=_mesh,
    scratch_types=[
        pltpu.VMEM((KJ, 128), jnp.int32),
        pltpu.VMEM((128,), f32),
        pltpu.VMEM((TROWS,), f32),
        pltpu.VMEM_SHARED((NROWS,), f32),
        pltpu.VMEM_SHARED((NROWS,), f32),
    ],
    compiler_params=_sc_params,
)
def _deg_kernel(src_dd, dst_dd, src_dp, dst_dp, o0, o1, o2, o3,
                idx_v, ones_v, zb, acc0, acc1):
    c = lax.axis_index("c")
    s = lax.axis_index("s")
    for i in range(8):
        ones_v[pl.ds(16 * i, 16)] = jnp.ones((16,), f32)

    def zbody(i, carry):
        zb[pl.ds(16 * i, 16)] = jnp.zeros((16,), f32)
        return carry

    lax.fori_loop(0, TROWS // 16, zbody, 0)
    base = s * TROWS
    pltpu.sync_copy(zb, acc0.at[pl.ds(base, TROWS)])
    pltpu.sync_copy(zb, acc1.at[pl.ds(base, TROWS)])
    plsc.subcore_barrier()

    def hist(idx_hbm, acc):
        pltpu.sync_copy(idx_hbm.at[s], idx_v)

        def body(j, carry):
            pltpu.sync_copy(ones_v, acc.at[idx_v.at[j]], add=True)
            return carry

        lax.fori_loop(0, KJ, body, 0)

    @pl.when(c == 0)
    def _():
        hist(src_dd, acc0)
        hist(dst_dd, acc1)

    @pl.when(c == 1)
    def _():
        hist(src_dp, acc0)
        hist(dst_dp, acc1)

    plsc.subcore_barrier()

    def emit(acc, out):
        pltpu.sync_copy(acc.at[pl.ds(base, TROWS)], zb)
        pltpu.sync_copy(zb, out.at[pl.ds(base, TROWS)])

    @pl.when(c == 0)
    def _():
        emit(acc0, o0)
        emit(acc1, o1)

    @pl.when(c == 1)
    def _():
        emit(acc0, o2)
        emit(acc1, o3)


@functools.partial(
    pl.kernel,
    out_type=[jax.ShapeDtypeStruct((NROWS, CW), f32)] * (2 * NCH),
    mesh=_mesh,
    scratch_types=[
        pltpu.VMEM((KJ, 128), jnp.int32),
        pltpu.VMEM((KJ, 128), jnp.int32),
        pltpu.VMEM((128, CW), f32),
        pltpu.VMEM((ZROWS, CW), f32),
        pltpu.VMEM((ZROWS, CW), f32),
        pltpu.VMEM_SHARED((NROWS, CW), f32),
        pltpu.SemaphoreType.DMA,
    ],
    compiler_params=_sc_params,
)
def _agg_kernel(*refs):
    ytabs = (refs[0:NCH], refs[NCH:2 * NCH])
    sdd, ddd, sdp, ddp = refs[2 * NCH:2 * NCH + 4]
    outs = (refs[2 * NCH + 4:3 * NCH + 4], refs[3 * NCH + 4:4 * NCH + 4])
    sidx_v, didx_v, rows_v, zb, bnc, acc, sem = refs[4 * NCH + 4:]
    srcs = (sdd, sdp)
    dsts = (ddd, ddp)

    c = lax.axis_index("c")
    s = lax.axis_index("s")

    def zbody(i, carry):
        zb[i, pl.ds(0, 16)] = jnp.zeros((16,), f32)
        return carry

    lax.fori_loop(0, ZROWS, zbody, 0)
    base = s * TROWS

    for rel in range(2):
        @pl.when(c == rel)
        def _(rel=rel):
            pltpu.sync_copy(srcs[rel].at[s], sidx_v)
            pltpu.sync_copy(dsts[rel].at[s], didx_v)
            for ch in range(NCH):
                for r in range(TROWS // ZROWS):
                    pltpu.sync_copy(zb, acc.at[pl.ds(base + ZROWS * r, ZROWS)])
                plsc.subcore_barrier()

                def body(j, carry, rel=rel, ch=ch):
                    pltpu.async_copy(
                        ytabs[rel][ch].at[sidx_v.at[j]], rows_v, sem
                    ).wait()
                    pltpu.sync_copy(rows_v, acc.at[didx_v.at[j]], add=True)
                    return carry

                lax.fori_loop(0, KJ, body, 0)
                plsc.subcore_barrier()
                for r in range(TROWS // ZROWS):
                    pltpu.sync_copy(
                        acc.at[pl.ds(base + ZROWS * r, ZROWS)], bnc)
                    pltpu.sync_copy(
                        bnc, outs[rel][ch].at[pl.ds(base + ZROWS * r, ZROWS)])


def _prescale_body(x_ref, odd_ref, odp_ref, *out_refs):
    x = x_ref[...]
    ns_dd = lax.rsqrt(jnp.maximum(odd_ref[...], 1.0))
    ns_dp = lax.rsqrt(jnp.maximum(odp_ref[...], 1.0))
    ydd = x * ns_dd
    ydp = x * ns_dp
    for ch in range(NCH):
        out_refs[ch][...] = ydd[:, CW * ch:CW * (ch + 1)]
        out_refs[NCH + ch][...] = ydp[:, CW * ch:CW * (ch + 1)]


def _prescale(xp, odeg_dd, odeg_dp):
    return pl.pallas_call(
        _prescale_body,
        grid=(GRID,),
        in_specs=[
            pl.BlockSpec((BR, D), lambda i: (i, 0)),
            pl.BlockSpec((BR, 1), lambda i: (i, 0)),
            pl.BlockSpec((BR, 1), lambda i: (i, 0)),
        ],
        out_specs=[pl.BlockSpec((BR, CW), lambda i: (i, 0))] * (2 * NCH),
        out_shape=[jax.ShapeDtypeStruct((NROWS, CW), f32)] * (2 * NCH),
    )(xp, odeg_dd, odeg_dp)


def _finish_body(*refs):
    dd_chunks = refs[0:NCH]
    dp_chunks = refs[NCH:2 * NCH]
    idd, idp, wdd, wdp, b, out_d, out_p = refs[2 * NCH:]
    bias = b[...]

    def one(chunks, deg_ref, w_ref, out_ref):
        agg = jnp.concatenate([r[...] for r in chunks], axis=1)
        h = jnp.dot(agg, w_ref[...], preferred_element_type=f32)
        h = h * lax.rsqrt(jnp.maximum(deg_ref[...], 1.0)) + bias
        nrm = jnp.sqrt(jnp.sum(h * h, axis=1, keepdims=True))
        out_ref[...] = h / jnp.maximum(nrm, 1e-12)

    one(dd_chunks, idd, wdd, out_d)
    one(dp_chunks, idp, wdp, out_p)


def _finish(aggs, ideg_dd, ideg_dp, W_dd, W_dp, bias2d):
    chunk_spec = pl.BlockSpec((BR, CW), lambda i: (i, 0))
    return pl.pallas_call(
        _finish_body,
        grid=(GRID,),
        in_specs=[chunk_spec] * (2 * NCH) + [
            pl.BlockSpec((BR, 1), lambda i: (i, 0)),
            pl.BlockSpec((BR, 1), lambda i: (i, 0)),
            pl.BlockSpec((D, D), lambda i: (0, 0)),
            pl.BlockSpec((D, D), lambda i: (0, 0)),
            pl.BlockSpec((1, D), lambda i: (0, 0)),
        ],
        out_specs=[pl.BlockSpec((BR, D), lambda i: (i, 0))] * 2,
        out_shape=[jax.ShapeDtypeStruct((NROWS, D), f32)] * 2,
    )(*aggs, ideg_dd, ideg_dp, W_dd, W_dp, bias2d)


def _prep_idx(idx):
    pad = jnp.full((EPAD - E,), DUMMY, dtype=jnp.int32)
    return jnp.concatenate([idx.astype(jnp.int32), pad]).reshape(NS, KJ, 128)


@jax.jit
def kernel(x_drug, x_protein, edge_index_dd, edge_index_dp, W_dd, W_dp, h_bias):
    del x_protein
    src_dd = _prep_idx(edge_index_dd[0])
    dst_dd = _prep_idx(edge_index_dd[1])
    src_dp = _prep_idx(edge_index_dp[0])
    dst_dp = _prep_idx(edge_index_dp[1])
    xp = jnp.concatenate([x_drug, jnp.zeros((NROWS - N, D), f32)])

    odeg_dd, ideg_dd, odeg_dp, ideg_dp = _deg_kernel(src_dd, dst_dd, src_dp, dst_dp)

    ytabs = _prescale(xp, odeg_dd.reshape(NROWS, 1), odeg_dp.reshape(NROWS, 1))
    aggs = _agg_kernel(*ytabs, src_dd, dst_dd, src_dp, dst_dp)

    out_d, out_p = _finish(
        aggs,
        ideg_dd.reshape(NROWS, 1),
        ideg_dp.reshape(NROWS, 1),
        W_dd, W_dp, h_bias.reshape(1, D),
    )
    return out_d[:N], out_p[:N]


# speedup vs baseline: 2.7295x; 1.1749x over previous
import functools

import jax
import jax.numpy as jnp
from jax import lax
from jax.experimental import pallas as pl
from jax.experimental.pallas import tpu as pltpu
from jax.experimental.pallas import tpu_sc as plsc

N = 50000
D = 128
E = 300000
NROWS = 50176
DUMMY = 50000
CW = 16
NCH = D // CW
NS = 16
KJ = 147
EPAD = NS * KJ * 128
ZROWS = 392
TROWS = NROWS // NS
BR = 512
GRID = NROWS // BR

_mesh = plsc.VectorSubcoreMesh(core_axis_name="c", subcore_axis_name="s")
_sc_params = pltpu.CompilerParams(use_tc_tiling_on_sc=False)
f32 = jnp.float32


@functools.partial(
    pl.kernel,
    out_type=[jax.ShapeDtypeStruct((NROWS,), f32)] * 4,
    mesh=_mesh,
    scratch_types=[
        pltpu.VMEM((KJ, 128), jnp.int32),
        pltpu.VMEM((128,), f32),
        pltpu.VMEM((TROWS,), f32),
        pltpu.VMEM_SHARED((NROWS,), f32),
        pltpu.VMEM_SHARED((NROWS,), f32),
    ],
    compiler_params=_sc_params,
)
def _deg_kernel(src_dd, dst_dd, src_dp, dst_dp, o0, o1, o2, o3,
                idx_v, ones_v, zb, acc0, acc1):
    c = lax.axis_index("c")
    s = lax.axis_index("s")
    for i in range(8):
        ones_v[pl.ds(16 * i, 16)] = jnp.ones((16,), f32)

    def zbody(i, carry):
        zb[pl.ds(16 * i, 16)] = jnp.zeros((16,), f32)
        return carry

    lax.fori_loop(0, TROWS // 16, zbody, 0)
    base = s * TROWS
    pltpu.sync_copy(zb, acc0.at[pl.ds(base, TROWS)])
    pltpu.sync_copy(zb, acc1.at[pl.ds(base, TROWS)])
    plsc.subcore_barrier()

    def hist(idx_hbm, acc):
        pltpu.sync_copy(idx_hbm.at[s], idx_v)

        def body(j, carry):
            pltpu.sync_copy(ones_v, acc.at[idx_v.at[j]], add=True)
            return carry

        lax.fori_loop(0, KJ, body, 0)

    @pl.when(c == 0)
    def _():
        hist(src_dd, acc0)
        hist(dst_dd, acc1)

    @pl.when(c == 1)
    def _():
        hist(src_dp, acc0)
        hist(dst_dp, acc1)

    plsc.subcore_barrier()

    def emit(acc, out):
        pltpu.sync_copy(acc.at[pl.ds(base, TROWS)], zb)
        pltpu.sync_copy(zb, out.at[pl.ds(base, TROWS)])

    @pl.when(c == 0)
    def _():
        emit(acc0, o0)
        emit(acc1, o1)

    @pl.when(c == 1)
    def _():
        emit(acc0, o2)
        emit(acc1, o3)


@functools.partial(
    pl.kernel,
    out_type=[jax.ShapeDtypeStruct((NROWS, NCH, CW), f32)] * 2,
    mesh=_mesh,
    scratch_types=[
        pltpu.VMEM((KJ, 128), jnp.int32),
        pltpu.VMEM((KJ, 128), jnp.int32),
        pltpu.VMEM((KJ, 128), jnp.int32),
        pltpu.VMEM((128, CW), f32),
        pltpu.VMEM((ZROWS, CW), f32),
        pltpu.VMEM((ZROWS, CW), f32),
        pltpu.VMEM_SHARED((NROWS, CW), f32),
        pltpu.SemaphoreType.DMA,
    ],
    compiler_params=_sc_params,
)
def _agg_kernel(y8dd, y8dp, sdd, ddd, sdp, ddp, out_dd, out_dp,
                sidx_v, didx_v, tidx_v, rows_v, zb, bnc, acc, sem):
    ytabs = (y8dd, y8dp)
    outs = (out_dd, out_dp)
    srcs = (sdd, sdp)
    dsts = (ddd, ddp)

    c = lax.axis_index("c")
    s = lax.axis_index("s")

    def zbody(i, carry):
        zb[i, pl.ds(0, 16)] = jnp.zeros((16,), f32)
        return carry

    lax.fori_loop(0, ZROWS, zbody, 0)
    base = s * TROWS

    for rel in range(2):
        @pl.when(c == rel)
        def _(rel=rel):
            pltpu.sync_copy(srcs[rel].at[s], sidx_v)
            pltpu.sync_copy(dsts[rel].at[s], didx_v)

            def mul8(j, carry):
                for k in range(8):
                    sl = pl.ds(16 * k, 16)
                    sidx_v[j, sl] = sidx_v[j, sl] * NCH
                return carry

            lax.fori_loop(0, KJ, mul8, 0)

            for ch in range(NCH):
                def addch(j, carry, ch=ch):
                    for k in range(8):
                        sl = pl.ds(16 * k, 16)
                        tidx_v[j, sl] = sidx_v[j, sl] + ch
                    return carry

                lax.fori_loop(0, KJ, addch, 0)
                for r in range(TROWS // ZROWS):
                    pltpu.sync_copy(zb, acc.at[pl.ds(base + ZROWS * r, ZROWS)])
                plsc.subcore_barrier()

                def body(j, carry, rel=rel):
                    pltpu.async_copy(
                        ytabs[rel].at[tidx_v.at[j]], rows_v, sem
                    ).wait()
                    pltpu.sync_copy(rows_v, acc.at[didx_v.at[j]], add=True)
                    return carry

                lax.fori_loop(0, KJ, body, 0)
                plsc.subcore_barrier()
                for r in range(TROWS // ZROWS):
                    pltpu.sync_copy(
                        acc.at[pl.ds(base + ZROWS * r, ZROWS)], bnc)
                    pltpu.sync_copy(
                        bnc, outs[rel].at[pl.ds(base + ZROWS * r, ZROWS), ch])


def _prescale_body(x_ref, odd_ref, odp_ref, ydd_ref, ydp_ref):
    x = x_ref[...]
    ydd_ref[...] = x * lax.rsqrt(jnp.maximum(odd_ref[...], 1.0))
    ydp_ref[...] = x * lax.rsqrt(jnp.maximum(odp_ref[...], 1.0))


def _prescale(xp, odeg_dd, odeg_dp):
    return pl.pallas_call(
        _prescale_body,
        grid=(GRID,),
        in_specs=[
            pl.BlockSpec((BR, D), lambda i: (i, 0)),
            pl.BlockSpec((BR, 1), lambda i: (i, 0)),
            pl.BlockSpec((BR, 1), lambda i: (i, 0)),
        ],
        out_specs=[pl.BlockSpec((BR, D), lambda i: (i, 0))] * 2,
        out_shape=[jax.ShapeDtypeStruct((NROWS, D), f32)] * 2,
    )(xp, odeg_dd, odeg_dp)


def _finish_body(add_ref, adp_ref, idd, idp, wdd, wdp, b, out_d, out_p):
    bias = b[...]

    def one(agg_ref, deg_ref, w_ref, out_ref):
        h = jnp.dot(agg_ref[...], w_ref[...], preferred_element_type=f32)
        h = h * lax.rsqrt(jnp.maximum(deg_ref[...], 1.0)) + bias
        nrm = jnp.sqrt(jnp.sum(h * h, axis=1, keepdims=True))
        out_ref[...] = h / jnp.maximum(nrm, 1e-12)

    one(add_ref, idd, wdd, out_d)
    one(adp_ref, idp, wdp, out_p)


def _finish(agg_dd, agg_dp, ideg_dd, ideg_dp, W_dd, W_dp, bias2d):
    return pl.pallas_call(
        _finish_body,
        grid=(GRID,),
        in_specs=[
            pl.BlockSpec((BR, D), lambda i: (i, 0)),
            pl.BlockSpec((BR, D), lambda i: (i, 0)),
            pl.BlockSpec((BR, 1), lambda i: (i, 0)),
            pl.BlockSpec((BR, 1), lambda i: (i, 0)),
            pl.BlockSpec((D, D), lambda i: (0, 0)),
            pl.BlockSpec((D, D), lambda i: (0, 0)),
            pl.BlockSpec((1, D), lambda i: (0, 0)),
        ],
        out_specs=[pl.BlockSpec((BR, D), lambda i: (i, 0))] * 2,
        out_shape=[jax.ShapeDtypeStruct((NROWS, D), f32)] * 2,
    )(agg_dd, agg_dp, ideg_dd, ideg_dp, W_dd, W_dp, bias2d)


def _prep_idx(idx):
    pad = jnp.full((EPAD - E,), DUMMY, dtype=jnp.int32)
    return jnp.concatenate([idx.astype(jnp.int32), pad]).reshape(NS, KJ, 128)


@jax.jit
def kernel(x_drug, x_protein, edge_index_dd, edge_index_dp, W_dd, W_dp, h_bias):
    del x_protein
    src_dd = _prep_idx(edge_index_dd[0])
    dst_dd = _prep_idx(edge_index_dd[1])
    src_dp = _prep_idx(edge_index_dp[0])
    dst_dp = _prep_idx(edge_index_dp[1])
    xp = jnp.concatenate([x_drug, jnp.zeros((NROWS - N, D), f32)])

    odeg_dd, ideg_dd, odeg_dp, ideg_dp = _deg_kernel(src_dd, dst_dd, src_dp, dst_dp)

    ydd, ydp = _prescale(xp, odeg_dd.reshape(NROWS, 1), odeg_dp.reshape(NROWS, 1))
    y8dd = ydd.reshape(NROWS * NCH, CW)
    y8dp = ydp.reshape(NROWS * NCH, CW)
    agg3dd, agg3dp = _agg_kernel(y8dd, y8dp, src_dd, dst_dd, src_dp, dst_dp)

    out_d, out_p = _finish(
        agg3dd.reshape(NROWS, D),
        agg3dp.reshape(NROWS, D),
        ideg_dd.reshape(NROWS, 1),
        ideg_dp.reshape(NROWS, 1),
        W_dd, W_dp, h_bias.reshape(1, D),
    )
    return out_d[:N], out_p[:N]


# speedup vs baseline: 5.0325x; 1.8437x over previous
import functools

import jax
import jax.numpy as jnp
from jax import lax
from jax.experimental import pallas as pl
from jax.experimental.pallas import tpu as pltpu
from jax.experimental.pallas import tpu_sc as plsc

N = 50000
D = 128
E = 300000
NROWS = 50176
DUMMY = 50000
CW = 16
NCH = D // CW
NS = 16
KJ = 147
EPAD = NS * KJ * 128
ZROWS = 392
TROWS = NROWS // NS
RING = 7
BR = 512
GRID = NROWS // BR

_mesh = plsc.VectorSubcoreMesh(core_axis_name="c", subcore_axis_name="s")
_sc_params = pltpu.CompilerParams(use_tc_tiling_on_sc=False)
f32 = jnp.float32


@functools.partial(
    pl.kernel,
    out_type=[jax.ShapeDtypeStruct((NROWS,), f32)] * 4,
    mesh=_mesh,
    scratch_types=[
        pltpu.VMEM((KJ, 128), jnp.int32),
        pltpu.VMEM((128,), f32),
        pltpu.VMEM((TROWS,), f32),
        pltpu.VMEM_SHARED((NROWS,), f32),
        pltpu.VMEM_SHARED((NROWS,), f32),
    ],
    compiler_params=_sc_params,
)
def _deg_kernel(src_dd, dst_dd, src_dp, dst_dp, o0, o1, o2, o3,
                idx_v, ones_v, zb, acc0, acc1):
    c = lax.axis_index("c")
    s = lax.axis_index("s")
    for i in range(8):
        ones_v[pl.ds(16 * i, 16)] = jnp.ones((16,), f32)

    def zbody(i, carry):
        zb[pl.ds(16 * i, 16)] = jnp.zeros((16,), f32)
        return carry

    lax.fori_loop(0, TROWS // 16, zbody, 0)
    base = s * TROWS
    pltpu.sync_copy(zb, acc0.at[pl.ds(base, TROWS)])
    pltpu.sync_copy(zb, acc1.at[pl.ds(base, TROWS)])
    plsc.subcore_barrier()

    def hist(idx_hbm, acc):
        pltpu.sync_copy(idx_hbm.at[s], idx_v)

        def body(j, carry):
            pltpu.sync_copy(ones_v, acc.at[idx_v.at[j]], add=True)
            return carry

        lax.fori_loop(0, KJ, body, 0)

    @pl.when(c == 0)
    def _():
        hist(src_dd, acc0)
        hist(dst_dd, acc1)

    @pl.when(c == 1)
    def _():
        hist(src_dp, acc0)
        hist(dst_dp, acc1)

    plsc.subcore_barrier()

    def emit(acc, out):
        pltpu.sync_copy(acc.at[pl.ds(base, TROWS)], zb)
        pltpu.sync_copy(zb, out.at[pl.ds(base, TROWS)])

    @pl.when(c == 0)
    def _():
        emit(acc0, o0)
        emit(acc1, o1)

    @pl.when(c == 1)
    def _():
        emit(acc0, o2)
        emit(acc1, o3)


@functools.partial(
    pl.kernel,
    out_type=[jax.ShapeDtypeStruct((NROWS, NCH, CW), f32)] * 2,
    mesh=_mesh,
    scratch_types=[
        pltpu.VMEM((KJ, 128), jnp.int32),
        pltpu.VMEM((KJ, 128), jnp.int32),
        pltpu.VMEM((ZROWS, CW), f32),
        pltpu.VMEM((ZROWS, CW), f32),
        pltpu.VMEM_SHARED((NROWS, CW), f32),
    ] + [pltpu.VMEM((128, CW), f32)] * RING
      + [pltpu.SemaphoreType.DMA] * (2 * RING),
    compiler_params=_sc_params,
)
def _agg_kernel(y8dd, y8dp, sdd, ddd, sdp, ddp, out_dd, out_dp,
                didx_v, tidx_v, zb, bnc, acc, *ring):
    rows = ring[:RING]
    gsem = ring[RING:2 * RING]
    ssem = ring[2 * RING:3 * RING]
    ytabs = (y8dd, y8dp)
    outs = (out_dd, out_dp)
    srcs = (sdd, sdp)
    dsts = (ddd, ddp)

    c = lax.axis_index("c")
    s = lax.axis_index("s")

    def zbody(i, carry):
        zb[i, pl.ds(0, 16)] = jnp.zeros((16,), f32)
        return carry

    lax.fori_loop(0, ZROWS, zbody, 0)
    base = s * TROWS

    for rel in range(2):
        @pl.when(c == rel)
        def _(rel=rel):
            pltpu.sync_copy(srcs[rel].at[s], tidx_v)
            pltpu.sync_copy(dsts[rel].at[s], didx_v)

            def mul8(j, carry):
                for k in range(8):
                    sl = pl.ds(16 * k, 16)
                    tidx_v[j, sl] = tidx_v[j, sl] * NCH
                return carry

            lax.fori_loop(0, KJ, mul8, 0)

            for ch in range(NCH):
                if ch > 0:
                    def add1(j, carry):
                        for k in range(8):
                            sl = pl.ds(16 * k, 16)
                            tidx_v[j, sl] = tidx_v[j, sl] + 1
                        return carry

                    lax.fori_loop(0, KJ, add1, 0)
                for r in range(TROWS // ZROWS):
                    pltpu.sync_copy(zb, acc.at[pl.ds(base + ZROWS * r, ZROWS)])
                plsc.subcore_barrier()

                yt = ytabs[rel]
                for t in range(RING):
                    pltpu.async_copy(yt.at[tidx_v.at[t]], rows[t], gsem[t])

                def body(k, carry, yt=yt):
                    for t in range(RING):
                        j = RING * k + t
                        pltpu.make_async_copy(
                            yt.at[tidx_v.at[0]], rows[t], gsem[t]).wait()
                        pltpu.async_copy(
                            rows[t], acc.at[didx_v.at[j]], ssem[t], add=True)
                        tp = (t + RING - 1) % RING
                        jp = j - 1

                        @pl.when((jp >= 0) & (jp + RING < KJ))
                        def _(tp=tp, jp=jp, yt=yt):
                            pltpu.make_async_copy(
                                rows[tp], acc.at[didx_v.at[0]], ssem[tp]).wait()
                            pltpu.async_copy(
                                yt.at[tidx_v.at[jp + RING]], rows[tp], gsem[tp])
                    return carry

                lax.fori_loop(0, KJ // RING, body, 0)
                for t in range(RING):
                    pltpu.make_async_copy(
                        rows[t], acc.at[didx_v.at[0]], ssem[t]).wait()
                plsc.subcore_barrier()
                for r in range(TROWS // ZROWS):
                    pltpu.sync_copy(
                        acc.at[pl.ds(base + ZROWS * r, ZROWS)], bnc)
                    pltpu.sync_copy(
                        bnc, outs[rel].at[pl.ds(base + ZROWS * r, ZROWS), ch])


def _prescale_body(x_ref, odd_ref, odp_ref, ydd_ref, ydp_ref):
    x = x_ref[...]
    ydd_ref[...] = x * lax.rsqrt(jnp.maximum(odd_ref[...], 1.0))
    ydp_ref[...] = x * lax.rsqrt(jnp.maximum(odp_ref[...], 1.0))


def _prescale(xp, odeg_dd, odeg_dp):
    return pl.pallas_call(
        _prescale_body,
        grid=(GRID,),
        in_specs=[
            pl.BlockSpec((BR, D), lambda i: (i, 0)),
            pl.BlockSpec((BR, 1), lambda i: (i, 0)),
            pl.BlockSpec((BR, 1), lambda i: (i, 0)),
        ],
        out_specs=[pl.BlockSpec((BR, D), lambda i: (i, 0))] * 2,
        out_shape=[jax.ShapeDtypeStruct((NROWS, D), f32)] * 2,
    )(xp, odeg_dd, odeg_dp)


def _finish_body(add_ref, adp_ref, idd, idp, wdd, wdp, b, out_d, out_p):
    bias = b[...]

    def one(agg_ref, deg_ref, w_ref, out_ref):
        h = jnp.dot(agg_ref[...], w_ref[...], preferred_element_type=f32)
        h = h * lax.rsqrt(jnp.maximum(deg_ref[...], 1.0)) + bias
        nrm = jnp.sqrt(jnp.sum(h * h, axis=1, keepdims=True))
        out_ref[...] = h / jnp.maximum(nrm, 1e-12)

    one(add_ref, idd, wdd, out_d)
    one(adp_ref, idp, wdp, out_p)


def _finish(agg_dd, agg_dp, ideg_dd, ideg_dp, W_dd, W_dp, bias2d):
    return pl.pallas_call(
        _finish_body,
        grid=(GRID,),
        in_specs=[
            pl.BlockSpec((BR, D), lambda i: (i, 0)),
            pl.BlockSpec((BR, D), lambda i: (i, 0)),
            pl.BlockSpec((BR, 1), lambda i: (i, 0)),
            pl.BlockSpec((BR, 1), lambda i: (i, 0)),
            pl.BlockSpec((D, D), lambda i: (0, 0)),
            pl.BlockSpec((D, D), lambda i: (0, 0)),
            pl.BlockSpec((1, D), lambda i: (0, 0)),
        ],
        out_specs=[pl.BlockSpec((BR, D), lambda i: (i, 0))] * 2,
        out_shape=[jax.ShapeDtypeStruct((NROWS, D), f32)] * 2,
    )(agg_dd, agg_dp, ideg_dd, ideg_dp, W_dd, W_dp, bias2d)


def _prep_idx(idx):
    pad = jnp.full((EPAD - E,), DUMMY, dtype=jnp.int32)
    return jnp.concatenate([idx.astype(jnp.int32), pad]).reshape(NS, KJ, 128)


@jax.jit
def kernel(x_drug, x_protein, edge_index_dd, edge_index_dp, W_dd, W_dp, h_bias):
    del x_protein
    src_dd = _prep_idx(edge_index_dd[0])
    dst_dd = _prep_idx(edge_index_dd[1])
    src_dp = _prep_idx(edge_index_dp[0])
    dst_dp = _prep_idx(edge_index_dp[1])
    xp = jnp.concatenate([x_drug, jnp.zeros((NROWS - N, D), f32)])

    odeg_dd, ideg_dd, odeg_dp, ideg_dp = _deg_kernel(src_dd, dst_dd, src_dp, dst_dp)

    ydd, ydp = _prescale(xp, odeg_dd.reshape(NROWS, 1), odeg_dp.reshape(NROWS, 1))
    y8dd = ydd.reshape(NROWS * NCH, CW)
    y8dp = ydp.reshape(NROWS * NCH, CW)
    agg3dd, agg3dp = _agg_kernel(y8dd, y8dp, src_dd, dst_dd, src_dp, dst_dp)

    out_d, out_p = _finish(
        agg3dd.reshape(NROWS, D),
        agg3dp.reshape(NROWS, D),
        ideg_dd.reshape(NROWS, 1),
        ideg_dp.reshape(NROWS, 1),
        W_dd, W_dp, h_bias.reshape(1, D),
    )
    return out_d[:N], out_p[:N]


# speedup vs baseline: 7.7684x; 1.5436x over previous
import functools

import jax
import jax.numpy as jnp
from jax import lax
from jax.experimental import pallas as pl
from jax.experimental.pallas import tpu as pltpu
from jax.experimental.pallas import tpu_sc as plsc

N = 50000
D = 128
E = 300000
NROWS = 50176
DUMMY = 50000
CW = 16
NCH = D // CW
NS = 16
KJ = 147
EPAD = NS * KJ * 128
ZROWS = 392
TROWS = NROWS // NS
RING = 7
BR = 512
GRID = NROWS // BR

_mesh = plsc.VectorSubcoreMesh(core_axis_name="c", subcore_axis_name="s")
_sc_params = pltpu.CompilerParams(use_tc_tiling_on_sc=False)
f32 = jnp.float32


@functools.partial(
    pl.kernel,
    out_type=[jax.ShapeDtypeStruct((NROWS,), f32)] * 4,
    mesh=_mesh,
    scratch_types=[
        pltpu.VMEM((KJ, 128), jnp.int32),
        pltpu.VMEM((128,), f32),
        pltpu.VMEM((TROWS,), f32),
        pltpu.VMEM_SHARED((NROWS,), f32),
        pltpu.VMEM_SHARED((NROWS,), f32),
    ],
    compiler_params=_sc_params,
)
def _deg_kernel(src_dd, dst_dd, src_dp, dst_dp, o0, o1, o2, o3,
                idx_v, ones_v, zb, acc0, acc1):
    c = lax.axis_index("c")
    s = lax.axis_index("s")
    for i in range(8):
        ones_v[pl.ds(16 * i, 16)] = jnp.ones((16,), f32)

    def zbody(i, carry):
        zb[pl.ds(16 * i, 16)] = jnp.zeros((16,), f32)
        return carry

    lax.fori_loop(0, TROWS // 16, zbody, 0)
    base = s * TROWS
    pltpu.sync_copy(zb, acc0.at[pl.ds(base, TROWS)])
    pltpu.sync_copy(zb, acc1.at[pl.ds(base, TROWS)])
    plsc.subcore_barrier()

    def hist(idx_hbm, acc):
        pltpu.sync_copy(idx_hbm.at[s], idx_v)

        def body(j, carry):
            pltpu.sync_copy(ones_v, acc.at[idx_v.at[j]], add=True)
            return carry

        lax.fori_loop(0, KJ, body, 0)

    @pl.when(c == 0)
    def _():
        hist(src_dd, acc0)
        hist(dst_dd, acc1)

    @pl.when(c == 1)
    def _():
        hist(src_dp, acc0)
        hist(dst_dp, acc1)

    plsc.subcore_barrier()

    def emit(acc, out):
        pltpu.sync_copy(acc.at[pl.ds(base, TROWS)], zb)
        pltpu.sync_copy(zb, out.at[pl.ds(base, TROWS)])

    @pl.when(c == 0)
    def _():
        emit(acc0, o0)
        emit(acc1, o1)

    @pl.when(c == 1)
    def _():
        emit(acc0, o2)
        emit(acc1, o3)


@functools.partial(
    pl.kernel,
    out_type=[jax.ShapeDtypeStruct((NROWS, D), f32)] * 2,
    mesh=_mesh,
    scratch_types=[
        pltpu.VMEM((KJ, 128), jnp.int32),
        pltpu.VMEM((KJ, 128), jnp.int32),
        pltpu.VMEM((ZROWS, CW), f32),
        pltpu.VMEM((ZROWS, CW), f32),
        pltpu.VMEM_SHARED((NROWS, CW), f32),
    ] + [pltpu.VMEM((128, CW), f32)] * RING
      + [pltpu.SemaphoreType.DMA] * (2 * RING),
    compiler_params=_sc_params,
)
def _agg_kernel(y8dd, y8dp, sdd, ddd, sdp, ddp, out_dd, out_dp,
                didx_v, tidx_v, zb, bnc, acc, *ring):
    rows = ring[:RING]
    gsem = ring[RING:2 * RING]
    ssem = ring[2 * RING:3 * RING]
    ytabs = (y8dd, y8dp)
    outs = (out_dd, out_dp)
    srcs = (sdd, sdp)
    dsts = (ddd, ddp)

    c = lax.axis_index("c")
    s = lax.axis_index("s")

    def zbody(i, carry):
        zb[i, pl.ds(0, 16)] = jnp.zeros((16,), f32)
        return carry

    lax.fori_loop(0, ZROWS, zbody, 0)
    base = s * TROWS

    for rel in range(2):
        @pl.when(c == rel)
        def _(rel=rel):
            pltpu.sync_copy(srcs[rel].at[s], tidx_v)
            pltpu.sync_copy(dsts[rel].at[s], didx_v)

            def mul8(j, carry):
                for k in range(8):
                    sl = pl.ds(16 * k, 16)
                    tidx_v[j, sl] = tidx_v[j, sl] * NCH
                return carry

            lax.fori_loop(0, KJ, mul8, 0)

            for ch in range(NCH):
                if ch > 0:
                    def add1(j, carry):
                        for k in range(8):
                            sl = pl.ds(16 * k, 16)
                            tidx_v[j, sl] = tidx_v[j, sl] + 1
                        return carry

                    lax.fori_loop(0, KJ, add1, 0)
                for r in range(TROWS // ZROWS):
                    pltpu.sync_copy(zb, acc.at[pl.ds(base + ZROWS * r, ZROWS)])
                plsc.subcore_barrier()

                yt = ytabs[rel]
                for t in range(RING):
                    pltpu.async_copy(yt.at[tidx_v.at[t]], rows[t], gsem[t])

                def body(k, carry, yt=yt):
                    for t in range(RING):
                        j = RING * k + t
                        pltpu.make_async_copy(
                            yt.at[tidx_v.at[0]], rows[t], gsem[t]).wait()
                        pltpu.async_copy(
                            rows[t], acc.at[didx_v.at[j]], ssem[t], add=True)
                        tp = (t + RING - 1) % RING
                        jp = j - 1

                        @pl.when((jp >= 0) & (jp + RING < KJ))
                        def _(tp=tp, jp=jp, yt=yt):
                            pltpu.make_async_copy(
                                rows[tp], acc.at[didx_v.at[0]], ssem[tp]).wait()
                            pltpu.async_copy(
                                yt.at[tidx_v.at[jp + RING]], rows[tp], gsem[tp])
                    return carry

                lax.fori_loop(0, KJ // RING, body, 0)
                for t in range(RING):
                    pltpu.make_async_copy(
                        rows[t], acc.at[didx_v.at[0]], ssem[t]).wait()
                plsc.subcore_barrier()
                for r in range(TROWS // ZROWS):
                    pltpu.sync_copy(
                        acc.at[pl.ds(base + ZROWS * r, ZROWS)], bnc)
                    pltpu.sync_copy(
                        bnc,
                        outs[rel].at[pl.ds(base + ZROWS * r, ZROWS),
                                     pl.ds(CW * ch, CW)])


def _prescale_body(x_ref, odd_ref, odp_ref, ydd_ref, ydp_ref):
    x = x_ref[...]
    ydd_ref[...] = x * lax.rsqrt(jnp.maximum(odd_ref[...], 1.0))
    ydp_ref[...] = x * lax.rsqrt(jnp.maximum(odp_ref[...], 1.0))


def _prescale(xp, odeg_dd, odeg_dp):
    return pl.pallas_call(
        _prescale_body,
        grid=(GRID,),
        in_specs=[
            pl.BlockSpec((BR, D), lambda i: (i, 0)),
            pl.BlockSpec((BR, 1), lambda i: (i, 0)),
            pl.BlockSpec((BR, 1), lambda i: (i, 0)),
        ],
        out_specs=[pl.BlockSpec((BR, D), lambda i: (i, 0))] * 2,
        out_shape=[jax.ShapeDtypeStruct((NROWS, D), f32)] * 2,
    )(xp, odeg_dd, odeg_dp)


def _finish_body(add_ref, adp_ref, idd, idp, wdd, wdp, b, out_d, out_p):
    bias = b[...]

    def one(agg_ref, deg_ref, w_ref, out_ref):
        h = jnp.dot(agg_ref[...], w_ref[...], preferred_element_type=f32)
        h = h * lax.rsqrt(jnp.maximum(deg_ref[...], 1.0)) + bias
        nrm = jnp.sqrt(jnp.sum(h * h, axis=1, keepdims=True))
        out_ref[...] = h / jnp.maximum(nrm, 1e-12)

    one(add_ref, idd, wdd, out_d)
    one(adp_ref, idp, wdp, out_p)


def _finish(agg_dd, agg_dp, ideg_dd, ideg_dp, W_dd, W_dp, bias2d):
    fbr = 400
    return pl.pallas_call(
        _finish_body,
        grid=(N // fbr,),
        in_specs=[
            pl.BlockSpec((fbr, D), lambda i: (i, 0)),
            pl.BlockSpec((fbr, D), lambda i: (i, 0)),
            pl.BlockSpec((fbr, 1), lambda i: (i, 0)),
            pl.BlockSpec((fbr, 1), lambda i: (i, 0)),
            pl.BlockSpec((D, D), lambda i: (0, 0)),
            pl.BlockSpec((D, D), lambda i: (0, 0)),
            pl.BlockSpec((1, D), lambda i: (0, 0)),
        ],
        out_specs=[pl.BlockSpec((fbr, D), lambda i: (i, 0))] * 2,
        out_shape=[jax.ShapeDtypeStruct((N, D), f32)] * 2,
    )(agg_dd, agg_dp, ideg_dd, ideg_dp, W_dd, W_dp, bias2d)


def _prep_idx(idx):
    pad = jnp.full((EPAD - E,), DUMMY, dtype=jnp.int32)
    return jnp.concatenate([idx.astype(jnp.int32), pad]).reshape(NS, KJ, 128)


@jax.jit
def kernel(x_drug, x_protein, edge_index_dd, edge_index_dp, W_dd, W_dp, h_bias):
    del x_protein
    src_dd = _prep_idx(edge_index_dd[0])
    dst_dd = _prep_idx(edge_index_dd[1])
    src_dp = _prep_idx(edge_index_dp[0])
    dst_dp = _prep_idx(edge_index_dp[1])
    xp = jnp.concatenate([x_drug, jnp.zeros((NROWS - N, D), f32)])

    odeg_dd, ideg_dd, odeg_dp, ideg_dp = _deg_kernel(src_dd, dst_dd, src_dp, dst_dp)

    ydd, ydp = _prescale(xp, odeg_dd.reshape(NROWS, 1), odeg_dp.reshape(NROWS, 1))
    y8dd = ydd.reshape(NROWS * NCH, CW)
    y8dp = ydp.reshape(NROWS * NCH, CW)
    agg_dd, agg_dp = _agg_kernel(y8dd, y8dp, src_dd, dst_dd, src_dp, dst_dp)

    out_d, out_p = _finish(
        agg_dd,
        agg_dp,
        ideg_dd.reshape(NROWS, 1),
        ideg_dp.reshape(NROWS, 1),
        W_dd, W_dp, h_bias.reshape(1, D),
    )
    return out_d, out_p


# speedup vs baseline: 8.2589x; 1.0631x over previous
import functools

import jax
import jax.numpy as jnp
from jax import lax
from jax.experimental import pallas as pl
from jax.experimental.pallas import tpu as pltpu
from jax.experimental.pallas import tpu_sc as plsc

N = 50000
D = 128
E = 300000
NROWS = 50176
DUMMY = 50000
CW = 16
NCH = D // CW
NS = 16
KJ = 147
EPAD = NS * KJ * 128
ZROWS = 392
TROWS = NROWS // NS
RING = 7
DELAY = 3
BR = 512
GRID = NROWS // BR

_mesh = plsc.VectorSubcoreMesh(core_axis_name="c", subcore_axis_name="s")
_sc_params = pltpu.CompilerParams(use_tc_tiling_on_sc=False)
f32 = jnp.float32


@functools.partial(
    pl.kernel,
    out_type=[jax.ShapeDtypeStruct((NROWS,), f32)] * 4,
    mesh=_mesh,
    scratch_types=[
        pltpu.VMEM((KJ, 128), jnp.int32),
        pltpu.VMEM((128,), f32),
        pltpu.VMEM((TROWS,), f32),
        pltpu.VMEM_SHARED((NROWS,), f32),
        pltpu.VMEM_SHARED((NROWS,), f32),
    ],
    compiler_params=_sc_params,
)
def _deg_kernel(src_dd, dst_dd, src_dp, dst_dp, o0, o1, o2, o3,
                idx_v, ones_v, zb, acc0, acc1):
    c = lax.axis_index("c")
    s = lax.axis_index("s")
    for i in range(8):
        ones_v[pl.ds(16 * i, 16)] = jnp.ones((16,), f32)

    def zbody(i, carry):
        zb[pl.ds(16 * i, 16)] = jnp.zeros((16,), f32)
        return carry

    lax.fori_loop(0, TROWS // 16, zbody, 0)
    base = s * TROWS
    pltpu.sync_copy(zb, acc0.at[pl.ds(base, TROWS)])
    pltpu.sync_copy(zb, acc1.at[pl.ds(base, TROWS)])
    plsc.subcore_barrier()

    def hist(idx_hbm, acc):
        pltpu.sync_copy(idx_hbm.at[s], idx_v)

        def body(j, carry):
            pltpu.sync_copy(ones_v, acc.at[idx_v.at[j]], add=True)
            return carry

        lax.fori_loop(0, KJ, body, 0)

    @pl.when(c == 0)
    def _():
        hist(src_dd, acc0)
        hist(dst_dd, acc1)

    @pl.when(c == 1)
    def _():
        hist(src_dp, acc0)
        hist(dst_dp, acc1)

    plsc.subcore_barrier()

    def emit(acc, out):
        pltpu.sync_copy(acc.at[pl.ds(base, TROWS)], zb)
        pltpu.sync_copy(zb, out.at[pl.ds(base, TROWS)])

    @pl.when(c == 0)
    def _():
        emit(acc0, o0)
        emit(acc1, o1)

    @pl.when(c == 1)
    def _():
        emit(acc0, o2)
        emit(acc1, o3)


@functools.partial(
    pl.kernel,
    out_type=[jax.ShapeDtypeStruct((NROWS, D), f32)] * 2,
    mesh=_mesh,
    scratch_types=[
        pltpu.VMEM((KJ, 128), jnp.int32),
        pltpu.VMEM((KJ, 128), jnp.int32),
        pltpu.VMEM((ZROWS, CW), f32),
        pltpu.VMEM((ZROWS, CW), f32),
        pltpu.VMEM_SHARED((NROWS, CW), f32),
    ] + [pltpu.VMEM((128, CW), f32)] * RING
      + [pltpu.SemaphoreType.DMA] * (2 * RING),
    compiler_params=_sc_params,
)
def _agg_kernel(y8dd, y8dp, sdd, ddd, sdp, ddp, out_dd, out_dp,
                didx_v, tidx_v, zb, bnc, acc, *ring):
    rows = ring[:RING]
    gsem = ring[RING:2 * RING]
    ssem = ring[2 * RING:3 * RING]
    ytabs = (y8dd, y8dp)
    outs = (out_dd, out_dp)
    srcs = (sdd, sdp)
    dsts = (ddd, ddp)

    c = lax.axis_index("c")
    s = lax.axis_index("s")

    def zbody(i, carry):
        zb[i, pl.ds(0, 16)] = jnp.zeros((16,), f32)
        return carry

    lax.fori_loop(0, ZROWS, zbody, 0)
    base = s * TROWS

    for rel in range(2):
        @pl.when(c == rel)
        def _(rel=rel):
            pltpu.sync_copy(srcs[rel].at[s], tidx_v)
            pltpu.sync_copy(dsts[rel].at[s], didx_v)

            def mul8(j, carry):
                for k in range(8):
                    sl = pl.ds(16 * k, 16)
                    tidx_v[j, sl] = tidx_v[j, sl] * NCH
                return carry

            lax.fori_loop(0, KJ, mul8, 0)

            for ch in range(NCH):
                if ch > 0:
                    def add1(j, carry):
                        for k in range(8):
                            sl = pl.ds(16 * k, 16)
                            tidx_v[j, sl] = tidx_v[j, sl] + 1
                        return carry

                    lax.fori_loop(0, KJ, add1, 0)
                for r in range(TROWS // ZROWS):
                    pltpu.sync_copy(zb, acc.at[pl.ds(base + ZROWS * r, ZROWS)])
                plsc.subcore_barrier()

                yt = ytabs[rel]
                for t in range(RING):
                    pltpu.async_copy(yt.at[tidx_v.at[t]], rows[t], gsem[t])

                def body(k, carry, yt=yt):
                    for t in range(RING):
                        j = RING * k + t
                        pltpu.make_async_copy(
                            yt.at[tidx_v.at[0]], rows[t], gsem[t]).wait()
                        pltpu.async_copy(
                            rows[t], acc.at[didx_v.at[j]], ssem[t], add=True)
                        tp = (t + RING - DELAY) % RING
                        jp = j - DELAY

                        @pl.when((jp >= 0) & (jp + RING < KJ))
                        def _(tp=tp, jp=jp, yt=yt):
                            pltpu.make_async_copy(
                                rows[tp], acc.at[didx_v.at[0]], ssem[tp]).wait()
                            pltpu.async_copy(
                                yt.at[tidx_v.at[jp + RING]], rows[tp], gsem[tp])
                    return carry

                lax.fori_loop(0, KJ // RING, body, 0)
                for t in range(RING):
                    pltpu.make_async_copy(
                        rows[t], acc.at[didx_v.at[0]], ssem[t]).wait()
                plsc.subcore_barrier()
                for r in range(TROWS // ZROWS):
                    pltpu.sync_copy(
                        acc.at[pl.ds(base + ZROWS * r, ZROWS)], bnc)
                    pltpu.sync_copy(
                        bnc,
                        outs[rel].at[pl.ds(base + ZROWS * r, ZROWS),
                                     pl.ds(CW * ch, CW)])


def _prescale_body(x_ref, odd_ref, odp_ref, ydd_ref, ydp_ref):
    x = x_ref[...]
    ydd_ref[...] = x * lax.rsqrt(jnp.maximum(odd_ref[...], 1.0))
    ydp_ref[...] = x * lax.rsqrt(jnp.maximum(odp_ref[...], 1.0))


def _prescale(xp, odeg_dd, odeg_dp):
    pbr = 1568
    return pl.pallas_call(
        _prescale_body,
        grid=(NROWS // pbr,),
        in_specs=[
            pl.BlockSpec((pbr, D), lambda i: (i, 0)),
            pl.BlockSpec((pbr, 1), lambda i: (i, 0)),
            pl.BlockSpec((pbr, 1), lambda i: (i, 0)),
        ],
        out_specs=[pl.BlockSpec((pbr, D), lambda i: (i, 0))] * 2,
        out_shape=[jax.ShapeDtypeStruct((NROWS, D), f32)] * 2,
    )(xp, odeg_dd, odeg_dp)


def _finish_body(add_ref, adp_ref, idd, idp, wdd, wdp, b, out_d, out_p):
    bias = b[...]

    def one(agg_ref, deg_ref, w_ref, out_ref):
        h = jnp.dot(agg_ref[...], w_ref[...], preferred_element_type=f32)
        h = h * lax.rsqrt(jnp.maximum(deg_ref[...], 1.0)) + bias
        nrm = jnp.sqrt(jnp.sum(h * h, axis=1, keepdims=True))
        out_ref[...] = h / jnp.maximum(nrm, 1e-12)

    one(add_ref, idd, wdd, out_d)
    one(adp_ref, idp, wdp, out_p)


def _finish(agg_dd, agg_dp, ideg_dd, ideg_dp, W_dd, W_dp, bias2d):
    fbr = 2000
    return pl.pallas_call(
        _finish_body,
        grid=(N // fbr,),
        in_specs=[
            pl.BlockSpec((fbr, D), lambda i: (i, 0)),
            pl.BlockSpec((fbr, D), lambda i: (i, 0)),
            pl.BlockSpec((fbr, 1), lambda i: (i, 0)),
            pl.BlockSpec((fbr, 1), lambda i: (i, 0)),
            pl.BlockSpec((D, D), lambda i: (0, 0)),
            pl.BlockSpec((D, D), lambda i: (0, 0)),
            pl.BlockSpec((1, D), lambda i: (0, 0)),
        ],
        out_specs=[pl.BlockSpec((fbr, D), lambda i: (i, 0))] * 2,
        out_shape=[jax.ShapeDtypeStruct((N, D), f32)] * 2,
    )(agg_dd, agg_dp, ideg_dd, ideg_dp, W_dd, W_dp, bias2d)


def _prep_idx(idx):
    pad = jnp.full((EPAD - E,), DUMMY, dtype=jnp.int32)
    return jnp.concatenate([idx.astype(jnp.int32), pad]).reshape(NS, KJ, 128)


@jax.jit
def kernel(x_drug, x_protein, edge_index_dd, edge_index_dp, W_dd, W_dp, h_bias):
    del x_protein
    src_dd = _prep_idx(edge_index_dd[0])
    dst_dd = _prep_idx(edge_index_dd[1])
    src_dp = _prep_idx(edge_index_dp[0])
    dst_dp = _prep_idx(edge_index_dp[1])
    xp = jnp.concatenate([x_drug, jnp.zeros((NROWS - N, D), f32)])

    odeg_dd, ideg_dd, odeg_dp, ideg_dp = _deg_kernel(src_dd, dst_dd, src_dp, dst_dp)

    ydd, ydp = _prescale(xp, odeg_dd.reshape(NROWS, 1), odeg_dp.reshape(NROWS, 1))
    y8dd = ydd.reshape(NROWS * NCH, CW)
    y8dp = ydp.reshape(NROWS * NCH, CW)
    agg_dd, agg_dp = _agg_kernel(y8dd, y8dp, src_dd, dst_dd, src_dp, dst_dp)

    out_d, out_p = _finish(
        agg_dd,
        agg_dp,
        ideg_dd.reshape(NROWS, 1),
        ideg_dp.reshape(NROWS, 1),
        W_dd, W_dp, h_bias.reshape(1, D),
    )
    return out_d, out_p


# speedup vs baseline: 9.0889x; 1.1005x over previous
import functools

import jax
import jax.numpy as jnp
from jax import lax
from jax.experimental import pallas as pl
from jax.experimental.pallas import tpu as pltpu
from jax.experimental.pallas import tpu_sc as plsc

N = 50000
D = 128
E = 300000
NROWS = 50176
DUMMY = 50000
CW = 16
NCH = D // CW
NS = 16
KJ = 147
EPAD = NS * KJ * 128
ZROWS = 392
TROWS = NROWS // NS
RING = 7
DELAY = 2
BR = 512
GRID = NROWS // BR

_mesh = plsc.VectorSubcoreMesh(core_axis_name="c", subcore_axis_name="s")
_sc_params = pltpu.CompilerParams(use_tc_tiling_on_sc=False)
f32 = jnp.float32


@functools.partial(
    pl.kernel,
    out_type=[jax.ShapeDtypeStruct((NROWS,), f32)] * 4,
    mesh=_mesh,
    scratch_types=[
        pltpu.VMEM((KJ, 128), jnp.int32),
        pltpu.VMEM((128,), f32),
        pltpu.VMEM((TROWS,), f32),
        pltpu.VMEM_SHARED((NROWS,), f32),
        pltpu.VMEM_SHARED((NROWS,), f32),
    ],
    compiler_params=_sc_params,
)
def _deg_kernel(src_dd, dst_dd, src_dp, dst_dp, o0, o1, o2, o3,
                idx_v, ones_v, zb, acc0, acc1):
    c = lax.axis_index("c")
    s = lax.axis_index("s")
    for i in range(8):
        ones_v[pl.ds(16 * i, 16)] = jnp.ones((16,), f32)

    def zbody(i, carry):
        zb[pl.ds(16 * i, 16)] = jnp.zeros((16,), f32)
        return carry

    lax.fori_loop(0, TROWS // 16, zbody, 0)
    base = s * TROWS
    pltpu.sync_copy(zb, acc0.at[pl.ds(base, TROWS)])
    pltpu.sync_copy(zb, acc1.at[pl.ds(base, TROWS)])
    plsc.subcore_barrier()

    def hist(idx_hbm, acc):
        pltpu.sync_copy(idx_hbm.at[s], idx_v)

        def body(j, carry):
            pltpu.sync_copy(ones_v, acc.at[idx_v.at[j]], add=True)
            return carry

        lax.fori_loop(0, KJ, body, 0)

    @pl.when(c == 0)
    def _():
        hist(src_dd, acc0)
        hist(dst_dd, acc1)

    @pl.when(c == 1)
    def _():
        hist(src_dp, acc0)
        hist(dst_dp, acc1)

    plsc.subcore_barrier()

    def emit(acc, out):
        pltpu.sync_copy(acc.at[pl.ds(base, TROWS)], zb)
        pltpu.sync_copy(zb, out.at[pl.ds(base, TROWS)])

    @pl.when(c == 0)
    def _():
        emit(acc0, o0)
        emit(acc1, o1)

    @pl.when(c == 1)
    def _():
        emit(acc0, o2)
        emit(acc1, o3)


@functools.partial(
    pl.kernel,
    out_type=[jax.ShapeDtypeStruct((NROWS, D), f32)] * 2,
    mesh=_mesh,
    scratch_types=[
        pltpu.VMEM((KJ, 128), jnp.int32),
        pltpu.VMEM((KJ, 128), jnp.int32),
        pltpu.VMEM((ZROWS, CW), f32),
        pltpu.VMEM((ZROWS, CW), f32),
        pltpu.VMEM((ZROWS, CW), f32),
        pltpu.VMEM_SHARED((NROWS, CW), f32),
    ] + [pltpu.VMEM((128, CW), f32)] * RING
      + [pltpu.SemaphoreType.DMA] * (2 * RING + 2),
    compiler_params=_sc_params,
)
def _agg_kernel(y8dd, y8dp, sdd, ddd, sdp, ddp, out_dd, out_dp,
                didx_v, tidx_v, zb, bnc, bnc2, acc, *ring):
    rows = ring[:RING]
    gsem = ring[RING:2 * RING]
    ssem = ring[2 * RING:3 * RING]
    wsem = ring[3 * RING:3 * RING + 2]
    bncs = (bnc, bnc2)
    ytabs = (y8dd, y8dp)
    outs = (out_dd, out_dp)
    srcs = (sdd, sdp)
    dsts = (ddd, ddp)

    c = lax.axis_index("c")
    s = lax.axis_index("s")

    def zbody(i, carry):
        zb[i, pl.ds(0, 16)] = jnp.zeros((16,), f32)
        return carry

    lax.fori_loop(0, ZROWS, zbody, 0)
    base = s * TROWS

    for rel in range(2):
        @pl.when(c == rel)
        def _(rel=rel):
            pltpu.sync_copy(srcs[rel].at[s], tidx_v)
            pltpu.sync_copy(dsts[rel].at[s], didx_v)

            def mul8(j, carry):
                for k in range(8):
                    sl = pl.ds(16 * k, 16)
                    tidx_v[j, sl] = tidx_v[j, sl] * NCH
                return carry

            lax.fori_loop(0, KJ, mul8, 0)

            for r in range(TROWS // ZROWS):
                pltpu.sync_copy(zb, acc.at[pl.ds(base + ZROWS * r, ZROWS)])

            for ch in range(NCH):
                if ch > 0:
                    def add1(j, carry):
                        for k in range(8):
                            sl = pl.ds(16 * k, 16)
                            tidx_v[j, sl] = tidx_v[j, sl] + 1
                        return carry

                    lax.fori_loop(0, KJ, add1, 0)
                plsc.subcore_barrier()

                yt = ytabs[rel]
                for t in range(RING):
                    pltpu.async_copy(yt.at[tidx_v.at[t]], rows[t], gsem[t])

                def body(k, carry, yt=yt):
                    for t in range(RING):
                        j = RING * k + t
                        pltpu.make_async_copy(
                            yt.at[tidx_v.at[0]], rows[t], gsem[t]).wait()
                        pltpu.async_copy(
                            rows[t], acc.at[didx_v.at[j]], ssem[t], add=True)
                        tp = (t + RING - DELAY) % RING
                        jp = j - DELAY

                        @pl.when((jp >= 0) & (jp + RING < KJ))
                        def _(tp=tp, jp=jp, yt=yt):
                            pltpu.make_async_copy(
                                rows[tp], acc.at[didx_v.at[0]], ssem[tp]).wait()
                            pltpu.async_copy(
                                yt.at[tidx_v.at[jp + RING]], rows[tp], gsem[tp])
                    return carry

                lax.fori_loop(0, KJ // RING, body, 0)
                for t in range(RING):
                    pltpu.make_async_copy(
                        rows[t], acc.at[didx_v.at[0]], ssem[t]).wait()
                plsc.subcore_barrier()
                for r in range(TROWS // ZROWS):
                    b = r % 2
                    rsl = pl.ds(base + ZROWS * r, ZROWS)
                    osl = outs[rel].at[rsl, pl.ds(CW * ch, CW)]
                    if r >= 2:
                        pltpu.make_async_copy(bncs[b], osl, wsem[b]).wait()
                    pltpu.sync_copy(acc.at[rsl], bncs[b])
                    pltpu.sync_copy(zb, acc.at[rsl])
                    pltpu.async_copy(bncs[b], osl, wsem[b])
                for b in range(2):
                    pltpu.make_async_copy(
                        bncs[b],
                        outs[rel].at[pl.ds(base, ZROWS), pl.ds(CW * ch, CW)],
                        wsem[b]).wait()


def _prescale_body(x_ref, odd_ref, odp_ref, ydd_ref, ydp_ref):
    x = x_ref[...]
    ydd_ref[...] = x * lax.rsqrt(jnp.maximum(odd_ref[...], 1.0))
    ydp_ref[...] = x * lax.rsqrt(jnp.maximum(odp_ref[...], 1.0))


def _prescale(xp, odeg_dd, odeg_dp):
    pbr = 1568
    return pl.pallas_call(
        _prescale_body,
        grid=(NROWS // pbr,),
        in_specs=[
            pl.BlockSpec((pbr, D), lambda i: (i, 0)),
            pl.BlockSpec((pbr, 1), lambda i: (i, 0)),
            pl.BlockSpec((pbr, 1), lambda i: (i, 0)),
        ],
        out_specs=[pl.BlockSpec((pbr, D), lambda i: (i, 0))] * 2,
        out_shape=[jax.ShapeDtypeStruct((NROWS, D), f32)] * 2,
    )(xp, odeg_dd, odeg_dp)


def _finish_body(add_ref, adp_ref, idd, idp, wdd, wdp, b, out_d, out_p):
    bias = b[...]

    def one(agg_ref, deg_ref, w_ref, out_ref):
        h = jnp.dot(agg_ref[...], w_ref[...], preferred_element_type=f32)
        h = h * lax.rsqrt(jnp.maximum(deg_ref[...], 1.0)) + bias
        nrm = jnp.sqrt(jnp.sum(h * h, axis=1, keepdims=True))
        out_ref[...] = h / jnp.maximum(nrm, 1e-12)

    one(add_ref, idd, wdd, out_d)
    one(adp_ref, idp, wdp, out_p)


def _finish(agg_dd, agg_dp, ideg_dd, ideg_dp, W_dd, W_dp, bias2d):
    fbr = 2000
    return pl.pallas_call(
        _finish_body,
        grid=(N // fbr,),
        in_specs=[
            pl.BlockSpec((fbr, D), lambda i: (i, 0)),
            pl.BlockSpec((fbr, D), lambda i: (i, 0)),
            pl.BlockSpec((fbr, 1), lambda i: (i, 0)),
            pl.BlockSpec((fbr, 1), lambda i: (i, 0)),
            pl.BlockSpec((D, D), lambda i: (0, 0)),
            pl.BlockSpec((D, D), lambda i: (0, 0)),
            pl.BlockSpec((1, D), lambda i: (0, 0)),
        ],
        out_specs=[pl.BlockSpec((fbr, D), lambda i: (i, 0))] * 2,
        out_shape=[jax.ShapeDtypeStruct((N, D), f32)] * 2,
    )(agg_dd, agg_dp, ideg_dd, ideg_dp, W_dd, W_dp, bias2d)


def _prep_idx(idx):
    pad = jnp.full((EPAD - E,), DUMMY, dtype=jnp.int32)
    return jnp.concatenate([idx.astype(jnp.int32), pad]).reshape(NS, KJ, 128)


@jax.jit
def kernel(x_drug, x_protein, edge_index_dd, edge_index_dp, W_dd, W_dp, h_bias):
    del x_protein
    src_dd = _prep_idx(edge_index_dd[0])
    dst_dd = _prep_idx(edge_index_dd[1])
    src_dp = _prep_idx(edge_index_dp[0])
    dst_dp = _prep_idx(edge_index_dp[1])
    xp = jnp.concatenate([x_drug, jnp.zeros((NROWS - N, D), f32)])

    odeg_dd, ideg_dd, odeg_dp, ideg_dp = _deg_kernel(src_dd, dst_dd, src_dp, dst_dp)

    ydd, ydp = _prescale(xp, odeg_dd.reshape(NROWS, 1), odeg_dp.reshape(NROWS, 1))
    y8dd = ydd.reshape(NROWS * NCH, CW)
    y8dp = ydp.reshape(NROWS * NCH, CW)
    agg_dd, agg_dp = _agg_kernel(y8dd, y8dp, src_dd, dst_dd, src_dp, dst_dp)

    out_d, out_p = _finish(
        agg_dd,
        agg_dp,
        ideg_dd.reshape(NROWS, 1),
        ideg_dp.reshape(NROWS, 1),
        W_dd, W_dp, h_bias.reshape(1, D),
    )
    return out_d, out_p


# speedup vs baseline: 10.0023x; 1.1005x over previous
import functools

import jax
import jax.numpy as jnp
from jax import lax
from jax.experimental import pallas as pl
from jax.experimental.pallas import tpu as pltpu
from jax.experimental.pallas import tpu_sc as plsc

N = 50000
D = 128
E = 300000
NROWS = 50176
DUMMY = 50000
CW = 16
NCH = D // CW
NS = 16
KJ = 147
EPAD = NS * KJ * 128
ZROWS = 392
TROWS = NROWS // NS
RING = 7
DELAY = 1
BR = 512
GRID = NROWS // BR

_mesh = plsc.VectorSubcoreMesh(core_axis_name="c", subcore_axis_name="s")
_sc_params = pltpu.CompilerParams(use_tc_tiling_on_sc=False)
f32 = jnp.float32


@functools.partial(
    pl.kernel,
    out_type=[jax.ShapeDtypeStruct((NROWS,), f32)] * 2,
    mesh=_mesh,
    scratch_types=[
        pltpu.VMEM((KJ, 128), jnp.int32),
        pltpu.VMEM((128,), f32),
        pltpu.VMEM((TROWS,), f32),
        pltpu.VMEM_SHARED((NROWS,), f32),
    ],
    compiler_params=_sc_params,
)
def _deg_kernel(idx_dd, idx_dp, o0, o1, idx_v, ones_v, zb, acc):
    c = lax.axis_index("c")
    s = lax.axis_index("s")
    for i in range(8):
        ones_v[pl.ds(16 * i, 16)] = jnp.ones((16,), f32)

    def zbody(i, carry):
        zb[pl.ds(16 * i, 16)] = jnp.zeros((16,), f32)
        return carry

    lax.fori_loop(0, TROWS // 16, zbody, 0)
    base = s * TROWS
    pltpu.sync_copy(zb, acc.at[pl.ds(base, TROWS)])
    plsc.subcore_barrier()

    def hist(idx_hbm):
        pltpu.sync_copy(idx_hbm.at[s], idx_v)

        def body(j, carry):
            pltpu.sync_copy(ones_v, acc.at[idx_v.at[j]], add=True)
            return carry

        lax.fori_loop(0, KJ, body, 0)

    @pl.when(c == 0)
    def _():
        hist(idx_dd)

    @pl.when(c == 1)
    def _():
        hist(idx_dp)

    plsc.subcore_barrier()

    def emit(out):
        pltpu.sync_copy(acc.at[pl.ds(base, TROWS)], zb)
        pltpu.sync_copy(zb, out.at[pl.ds(base, TROWS)])

    @pl.when(c == 0)
    def _():
        emit(o0)

    @pl.when(c == 1)
    def _():
        emit(o1)


@functools.partial(
    pl.kernel,
    out_type=[jax.ShapeDtypeStruct((NROWS, D), f32)] * 2,
    mesh=_mesh,
    scratch_types=[
        pltpu.VMEM((KJ, 128), jnp.int32),
        pltpu.VMEM((KJ, 128), jnp.int32),
        pltpu.VMEM((ZROWS, CW), f32),
        pltpu.VMEM((ZROWS, CW), f32),
        pltpu.VMEM((ZROWS, CW), f32),
        pltpu.VMEM_SHARED((NROWS, CW), f32),
    ] + [pltpu.VMEM((128, CW), f32)] * RING
      + [pltpu.SemaphoreType.DMA] * (2 * RING + 2),
    compiler_params=_sc_params,
)
def _agg_kernel(y8dd, y8dp, sdd, ddd, sdp, ddp, out_dd, out_dp,
                didx_v, tidx_v, zb, bnc, bnc2, acc, *ring):
    rows = ring[:RING]
    gsem = ring[RING:2 * RING]
    ssem = ring[2 * RING:3 * RING]
    wsem = ring[3 * RING:3 * RING + 2]
    bncs = (bnc, bnc2)
    ytabs = (y8dd, y8dp)
    outs = (out_dd, out_dp)
    srcs = (sdd, sdp)
    dsts = (ddd, ddp)

    c = lax.axis_index("c")
    s = lax.axis_index("s")

    def zbody(i, carry):
        zb[i, pl.ds(0, 16)] = jnp.zeros((16,), f32)
        return carry

    lax.fori_loop(0, ZROWS, zbody, 0)
    base = s * TROWS

    for rel in range(2):
        @pl.when(c == rel)
        def _(rel=rel):
            pltpu.sync_copy(srcs[rel].at[s], tidx_v)
            pltpu.sync_copy(dsts[rel].at[s], didx_v)

            def mul8(j, carry):
                for k in range(8):
                    sl = pl.ds(16 * k, 16)
                    tidx_v[j, sl] = tidx_v[j, sl] * NCH
                return carry

            lax.fori_loop(0, KJ, mul8, 0)

            for r in range(TROWS // ZROWS):
                pltpu.sync_copy(zb, acc.at[pl.ds(base + ZROWS * r, ZROWS)])

            for ch in range(NCH):
                if ch > 0:
                    def add1(j, carry):
                        for k in range(8):
                            sl = pl.ds(16 * k, 16)
                            tidx_v[j, sl] = tidx_v[j, sl] + 1
                        return carry

                    lax.fori_loop(0, KJ, add1, 0)
                plsc.subcore_barrier()

                yt = ytabs[rel]
                for t in range(RING):
                    pltpu.async_copy(yt.at[tidx_v.at[t]], rows[t], gsem[t])

                def body(k, carry, yt=yt):
                    for t in range(RING):
                        j = RING * k + t
                        pltpu.make_async_copy(
                            yt.at[tidx_v.at[0]], rows[t], gsem[t]).wait()
                        pltpu.async_copy(
                            rows[t], acc.at[didx_v.at[j]], ssem[t], add=True)
                        tp = (t + RING - DELAY) % RING
                        jp = j - DELAY

                        @pl.when((jp >= 0) & (jp + RING < KJ))
                        def _(tp=tp, jp=jp, yt=yt):
                            pltpu.make_async_copy(
                                rows[tp], acc.at[didx_v.at[0]], ssem[tp]).wait()
                            pltpu.async_copy(
                                yt.at[tidx_v.at[jp + RING]], rows[tp], gsem[tp])
                    return carry

                lax.fori_loop(0, KJ // RING, body, 0)
                for t in range(RING):
                    pltpu.make_async_copy(
                        rows[t], acc.at[didx_v.at[0]], ssem[t]).wait()
                plsc.subcore_barrier()
                for r in range(TROWS // ZROWS):
                    b = r % 2
                    rsl = pl.ds(base + ZROWS * r, ZROWS)
                    osl = outs[rel].at[rsl, pl.ds(CW * ch, CW)]
                    if r >= 2:
                        pltpu.make_async_copy(bncs[b], osl, wsem[b]).wait()
                    pltpu.sync_copy(acc.at[rsl], bncs[b])
                    pltpu.sync_copy(zb, acc.at[rsl])
                    pltpu.async_copy(bncs[b], osl, wsem[b])
                for b in range(2):
                    pltpu.make_async_copy(
                        bncs[b],
                        outs[rel].at[pl.ds(base, ZROWS), pl.ds(CW * ch, CW)],
                        wsem[b]).wait()


def _prescale_body(x_ref, odd_ref, odp_ref, ydd_ref, ydp_ref):
    x = x_ref[...]
    ydd_ref[...] = x * lax.rsqrt(jnp.maximum(odd_ref[...], 1.0))
    ydp_ref[...] = x * lax.rsqrt(jnp.maximum(odp_ref[...], 1.0))


def _prescale(xp, odeg_dd, odeg_dp):
    pbr = 1568
    return pl.pallas_call(
        _prescale_body,
        grid=(NROWS // pbr,),
        in_specs=[
            pl.BlockSpec((pbr, D), lambda i: (i, 0)),
            pl.BlockSpec((pbr, 1), lambda i: (i, 0)),
            pl.BlockSpec((pbr, 1), lambda i: (i, 0)),
        ],
        out_specs=[pl.BlockSpec((pbr, D), lambda i: (i, 0))] * 2,
        out_shape=[jax.ShapeDtypeStruct((NROWS, D), f32)] * 2,
    )(xp, odeg_dd, odeg_dp)


def _finish_body(add_ref, adp_ref, idd, idp, wdd, wdp, b, out_d, out_p):
    bias = b[...]

    def one(agg_ref, deg_ref, w_ref, out_ref):
        h = jnp.dot(agg_ref[...], w_ref[...], preferred_element_type=f32)
        h = h * lax.rsqrt(jnp.maximum(deg_ref[...], 1.0)) + bias
        nrm = jnp.sqrt(jnp.sum(h * h, axis=1, keepdims=True))
        out_ref[...] = h / jnp.maximum(nrm, 1e-12)

    one(add_ref, idd, wdd, out_d)
    one(adp_ref, idp, wdp, out_p)


def _finish(agg_dd, agg_dp, ideg_dd, ideg_dp, W_dd, W_dp, bias2d):
    fbr = 2000
    return pl.pallas_call(
        _finish_body,
        grid=(N // fbr,),
        in_specs=[
            pl.BlockSpec((fbr, D), lambda i: (i, 0)),
            pl.BlockSpec((fbr, D), lambda i: (i, 0)),
            pl.BlockSpec((fbr, 1), lambda i: (i, 0)),
            pl.BlockSpec((fbr, 1), lambda i: (i, 0)),
            pl.BlockSpec((D, D), lambda i: (0, 0)),
            pl.BlockSpec((D, D), lambda i: (0, 0)),
            pl.BlockSpec((1, D), lambda i: (0, 0)),
        ],
        out_specs=[pl.BlockSpec((fbr, D), lambda i: (i, 0))] * 2,
        out_shape=[jax.ShapeDtypeStruct((N, D), f32)] * 2,
    )(agg_dd, agg_dp, ideg_dd, ideg_dp, W_dd, W_dp, bias2d)


def _prep_idx(idx):
    pad = jnp.full((EPAD - E,), DUMMY, dtype=jnp.int32)
    return jnp.concatenate([idx.astype(jnp.int32), pad]).reshape(NS, KJ, 128)


@jax.jit
def kernel(x_drug, x_protein, edge_index_dd, edge_index_dp, W_dd, W_dp, h_bias):
    del x_protein
    src_dd = _prep_idx(edge_index_dd[0])
    dst_dd = _prep_idx(edge_index_dd[1])
    src_dp = _prep_idx(edge_index_dp[0])
    dst_dp = _prep_idx(edge_index_dp[1])
    xp = jnp.concatenate([x_drug, jnp.zeros((NROWS - N, D), f32)])

    odeg_dd, odeg_dp = _deg_kernel(src_dd, src_dp)
    ideg_dd, ideg_dp = _deg_kernel(dst_dd, dst_dp)

    ydd, ydp = _prescale(xp, odeg_dd.reshape(NROWS, 1), odeg_dp.reshape(NROWS, 1))
    y8dd = ydd.reshape(NROWS * NCH, CW)
    y8dp = ydp.reshape(NROWS * NCH, CW)
    agg_dd, agg_dp = _agg_kernel(y8dd, y8dp, src_dd, dst_dd, src_dp, dst_dp)

    out_d, out_p = _finish(
        agg_dd,
        agg_dp,
        ideg_dd.reshape(NROWS, 1),
        ideg_dp.reshape(NROWS, 1),
        W_dd, W_dp, h_bias.reshape(1, D),
    )
    return out_d, out_p
